# baseline (speedup 1.0000x reference)
import numpy as np
import jax
import jax.numpy as jnp
from jax.experimental import pallas as pl


def _line_constants(num_rows: int):
    rng = np.random.default_rng(0)
    rand_row = int(rng.integers(0, num_rows))
    x_min, alpha = 1.0728769e-07, 1.0868737
    r = float(rng.random())
    relative_error = x_min * (1.0 - r) ** (-1.0 / (alpha - 1.0))
    return rand_row, relative_error


def kernel(forward_input):
    n_rows, n_cols = forward_input.shape
    rand_row, rel_err = _line_constants(n_rows)

    block_rows = 512
    grid = n_rows // block_rows

    def body(x_ref, o_ref):
        i = pl.program_id(0)
        row_ids = i * block_rows + jax.lax.broadcasted_iota(
            jnp.int32, (block_rows, 1), 0
        )
        scale = jnp.where(row_ids == rand_row,
                          jnp.float32(rel_err), jnp.float32(1.0))
        o_ref[...] = x_ref[...] * scale

    return pl.pallas_call(
        body,
        grid=(grid,),
        in_specs=[pl.BlockSpec((block_rows, n_cols), lambda i: (i, 0))],
        out_specs=pl.BlockSpec((block_rows, n_cols), lambda i: (i, 0)),
        out_shape=jax.ShapeDtypeStruct((n_rows, n_cols), forward_input.dtype),
    )(forward_input)
